# baseline (speedup 1.0000x reference)
import functools

import jax
import jax.numpy as jnp
from jax import lax
from jax.experimental import pallas as pl
from jax.experimental.pallas import tpu as pltpu
from jax.experimental.pallas import tpu_sc as plsc

N = 10000
E = 160000
D = 256
DH = 128
NC, NS = 2, 16
NPAD = 10240
PAD_DST = 10048
EPAD = 163840
EPT = EPAD // NS
NCH = EPT // 128
CH = 5
DEG_NCH = EPAD // 32 // 128
STRIPE = NPAD // NS

_mesh = plsc.VectorSubcoreMesh(core_axis_name="c", subcore_axis_name="s")


@functools.partial(
    pl.kernel,
    out_type=jax.ShapeDtypeStruct((NC, NPAD), jnp.float32),
    mesh=_mesh,
    scratch_types=[
        pltpu.VMEM((DEG_NCH, 128), jnp.int32),
        pltpu.VMEM((128,), jnp.float32),
        pltpu.VMEM_SHARED((NPAD,), jnp.float32),
    ],
)
def _sc_degree(dst_hbm, z_hbm, ones_hbm, out_hbm, didx, ones_v, acc):
    c = lax.axis_index("c")
    s = lax.axis_index("s")
    tid = c * NS + s
    pltpu.sync_copy(z_hbm.at[pl.ds(s * STRIPE, STRIPE)],
                    acc.at[pl.ds(s * STRIPE, STRIPE)])
    pltpu.sync_copy(dst_hbm.at[pl.ds(tid * DEG_NCH, DEG_NCH)], didx)
    pltpu.sync_copy(ones_hbm, ones_v)
    plsc.subcore_barrier()

    @pl.loop(0, DEG_NCH)
    def _(j):
        pltpu.sync_copy(ones_v, acc.at[didx.at[j]], add=True)

    plsc.subcore_barrier()
    pltpu.sync_copy(acc.at[pl.ds(s * STRIPE, STRIPE)],
                    out_hbm.at[c].at[pl.ds(s * STRIPE, STRIPE)])


@functools.partial(
    pl.kernel,
    out_type=jax.ShapeDtypeStruct((NC, NPAD, DH), jnp.float32),
    mesh=_mesh,
    scratch_types=[
        pltpu.VMEM((EPT,), jnp.int32),
        pltpu.VMEM((NCH * 2, 64), jnp.int32),
        pltpu.VMEM((64, DH), jnp.float32),
        pltpu.VMEM((64, DH), jnp.float32),
        pltpu.VMEM_SHARED((NPAD, DH), jnp.float32),
        pltpu.SemaphoreType.DMA,
        pltpu.SemaphoreType.DMA,
    ],
)
def _sc_spmm(u_hbm, src_hbm, dst_hbm, z_hbm, dummy_hbm, out_hbm, sidx, didx,
             gbufa, gbufb, acc, gsema, gsemb):
    c = lax.axis_index("c")
    s = lax.axis_index("s")
    @pl.loop(0, STRIPE // 128)
    def _(r):
        pltpu.sync_copy(z_hbm.at[pl.ds(s * STRIPE + r * 128, 128)],
                        acc.at[pl.ds(s * STRIPE + r * 128, 128)])

    pltpu.sync_copy(src_hbm.at[pl.ds(s * EPT, EPT)], sidx)
    pltpu.sync_copy(dst_hbm.at[pl.ds(s * NCH * 2, NCH * 2)], didx)
    plsc.subcore_barrier()

    @pl.loop(0, NCH * 2)
    def _(j):
        pltpu.sync_copy(u_hbm.at[c].at[sidx.at[pl.ds(j * 64, 64)]], gbufa)
        pltpu.sync_copy(gbufa, acc.at[didx.at[j]], add=True)

    plsc.subcore_barrier()

    @pl.loop(0, STRIPE // 128)
    def _(r):
        pltpu.sync_copy(acc.at[pl.ds(s * STRIPE + r * 128, 128)],
                        out_hbm.at[c].at[pl.ds(s * STRIPE + r * 128, 128)])


_R = 1000


def _tc_isd(degp):
    def body(d_ref, o_ref):
        d = d_ref[0:1, :N] + d_ref[1:2, :N]
        o_ref[...] = lax.rsqrt(jnp.maximum(d, 1.0))

    return pl.pallas_call(
        body,
        out_shape=jax.ShapeDtypeStruct((1, N), jnp.float32),
    )(degp)


def _tc_in(x, isd, W):
    def body(x_ref, i_ref, w_ref, o_ref):
        p = x_ref[...] * i_ref[...]
        ub = jnp.dot(p, w_ref[...], preferred_element_type=jnp.float32)
        o_ref[...] = jnp.stack([ub[:, :DH], ub[:, DH:]])

    return pl.pallas_call(
        body,
        grid=(N // _R,),
        in_specs=[
            pl.BlockSpec((_R, D), lambda i: (i, 0)),
            pl.BlockSpec((_R, 1), lambda i: (i, 0)),
            pl.BlockSpec((D, D), lambda i: (0, 0)),
        ],
        out_specs=pl.BlockSpec((NC, _R, DH), lambda i: (0, i, 0)),
        out_shape=jax.ShapeDtypeStruct((NC, N, DH), jnp.float32),
    )(x, isd, W)


def _tc_mid(v, isd, b, W):
    def body(v_ref, i_ref, b_ref, w_ref, o_ref):
        vb = v_ref[...]
        hcat = jnp.concatenate([vb[0], vb[1]], axis=1)
        h = jnp.maximum(hcat * i_ref[...] + b_ref[...], 0.0)
        p = h * i_ref[...]
        ub = jnp.dot(p, w_ref[...], preferred_element_type=jnp.float32)
        o_ref[...] = jnp.stack([ub[:, :DH], ub[:, DH:]])

    return pl.pallas_call(
        body,
        grid=(N // _R,),
        in_specs=[
            pl.BlockSpec((NC, _R, DH), lambda i: (0, i, 0)),
            pl.BlockSpec((_R, 1), lambda i: (i, 0)),
            pl.BlockSpec((1, D), lambda i: (0, 0)),
            pl.BlockSpec((D, D), lambda i: (0, 0)),
        ],
        out_specs=pl.BlockSpec((NC, _R, DH), lambda i: (0, i, 0)),
        out_shape=jax.ShapeDtypeStruct((NC, N, DH), jnp.float32),
    )(v, isd, b, W)


def _tc_out(v, isd, b):
    def body(v_ref, i_ref, b_ref, o_ref):
        vb = v_ref[...]
        hcat = jnp.concatenate([vb[0], vb[1]], axis=1)
        o_ref[...] = jnp.maximum(hcat * i_ref[...] + b_ref[...], 0.0)

    return pl.pallas_call(
        body,
        grid=(N // _R,),
        in_specs=[
            pl.BlockSpec((NC, _R, DH), lambda i: (0, i, 0)),
            pl.BlockSpec((_R, 1), lambda i: (i, 0)),
            pl.BlockSpec((1, D), lambda i: (0, 0)),
        ],
        out_specs=pl.BlockSpec((_R, D), lambda i: (i, 0)),
        out_shape=jax.ShapeDtypeStruct((N, D), jnp.float32),
    )(v, isd, b)


@jax.jit
def kernel(x, edge_index, W1, b1, W2, b2, W3, b3):
    src = edge_index[0]
    dst = edge_index[1]
    srcp = jnp.concatenate([src, jnp.zeros((EPAD - E,), jnp.int32)])
    dstp = jnp.concatenate([dst, jnp.full((EPAD - E,), PAD_DST, jnp.int32)])
    src2d = srcp.reshape(EPAD // 128, 128)
    dst2d = dstp.reshape(EPAD // 128, 128)
    dst2d64 = dstp.reshape(EPAD // 64, 64)
    z1 = jnp.zeros((NPAD,), jnp.float32)
    z2 = jnp.zeros((NPAD, DH), jnp.float32)
    ones2d = jnp.ones((128,), jnp.float32)
    dummy = jnp.zeros((128, DH), jnp.float32)

    degp = _sc_degree(dst2d, z1, ones2d)
    isd = _tc_isd(degp).reshape(N, 1)

    u = _tc_in(x, isd, W1)
    v = _sc_spmm(u, srcp, dst2d64, z2, dummy)
    u = _tc_mid(v, isd, b1.reshape(1, D), W2)
    v = _sc_spmm(u, srcp, dst2d64, z2, dummy)
    u = _tc_mid(v, isd, b2.reshape(1, D), W3)
    v = _sc_spmm(u, srcp, dst2d64, z2, dummy)
    return _tc_out(v, isd, b3.reshape(1, D))


# speedup vs baseline: 1.1229x; 1.1229x over previous
import functools

import jax
import jax.numpy as jnp
from jax import lax
from jax.experimental import pallas as pl
from jax.experimental.pallas import tpu as pltpu
from jax.experimental.pallas import tpu_sc as plsc

N = 10000
E = 160000
D = 256
DH = 128
NC, NS = 2, 16
NPAD = 10240
PAD_DST = 10048
EPAD = 163840
EPT = EPAD // NS
NCH = EPT // 128
CH = 5
DEG_NCH = EPAD // 32 // 128
STRIPE = NPAD // NS

_mesh = plsc.VectorSubcoreMesh(core_axis_name="c", subcore_axis_name="s")


@functools.partial(
    pl.kernel,
    out_type=jax.ShapeDtypeStruct((NC, NPAD), jnp.float32),
    mesh=_mesh,
    scratch_types=[
        pltpu.VMEM((DEG_NCH, 128), jnp.int32),
        pltpu.VMEM((128,), jnp.float32),
        pltpu.VMEM_SHARED((NPAD,), jnp.float32),
    ],
)
def _sc_degree(dst_hbm, z_hbm, ones_hbm, out_hbm, didx, ones_v, acc):
    c = lax.axis_index("c")
    s = lax.axis_index("s")
    tid = c * NS + s
    pltpu.sync_copy(z_hbm.at[pl.ds(s * STRIPE, STRIPE)],
                    acc.at[pl.ds(s * STRIPE, STRIPE)])
    pltpu.sync_copy(dst_hbm.at[pl.ds(tid * DEG_NCH, DEG_NCH)], didx)
    pltpu.sync_copy(ones_hbm, ones_v)
    plsc.subcore_barrier()

    @pl.loop(0, DEG_NCH)
    def _(j):
        pltpu.sync_copy(ones_v, acc.at[didx.at[j]], add=True)

    plsc.subcore_barrier()
    pltpu.sync_copy(acc.at[pl.ds(s * STRIPE, STRIPE)],
                    out_hbm.at[c].at[pl.ds(s * STRIPE, STRIPE)])


@functools.partial(
    pl.kernel,
    out_type=jax.ShapeDtypeStruct((NC, NPAD, DH), jnp.float32),
    mesh=_mesh,
    scratch_types=[
        pltpu.VMEM((EPT,), jnp.int32),
        pltpu.VMEM((NCH, 128), jnp.int32),
        pltpu.VMEM((128, DH), jnp.float32),
        pltpu.VMEM_SHARED((NPAD, DH), jnp.float32),
    ],
)
def _sc_spmm(u_hbm, src_hbm, dst_hbm, z_hbm, out_hbm, sidx, didx, gbuf, acc):
    c = lax.axis_index("c")
    s = lax.axis_index("s")
    @pl.loop(0, STRIPE // 128)
    def _(r):
        pltpu.sync_copy(z_hbm.at[pl.ds(s * STRIPE + r * 128, 128)],
                        acc.at[pl.ds(s * STRIPE + r * 128, 128)])

    pltpu.sync_copy(src_hbm.at[pl.ds(s * EPT, EPT)], sidx)
    pltpu.sync_copy(dst_hbm.at[pl.ds(s * NCH, NCH)], didx)
    plsc.subcore_barrier()

    @pl.loop(0, NCH)
    def _(j):
        pltpu.sync_copy(u_hbm.at[c].at[sidx.at[pl.ds(j * 128, 128)]], gbuf)
        pltpu.sync_copy(gbuf, acc.at[didx.at[j]], add=True)

    plsc.subcore_barrier()

    @pl.loop(0, STRIPE // 128)
    def _(r):
        pltpu.sync_copy(acc.at[pl.ds(s * STRIPE + r * 128, 128)],
                        out_hbm.at[c].at[pl.ds(s * STRIPE + r * 128, 128)])


_R = 1000


def _tc_isd(degp):
    def body(d_ref, o_ref):
        d = d_ref[0:1, :N] + d_ref[1:2, :N]
        o_ref[...] = lax.rsqrt(jnp.maximum(d, 1.0))

    return pl.pallas_call(
        body,
        out_shape=jax.ShapeDtypeStruct((1, N), jnp.float32),
    )(degp)


def _tc_in(x, isd, W):
    def body(x_ref, i_ref, w_ref, o_ref):
        p = x_ref[...] * i_ref[...]
        ub = jnp.dot(p, w_ref[...], preferred_element_type=jnp.float32)
        o_ref[...] = jnp.stack([ub[:, :DH], ub[:, DH:]])

    return pl.pallas_call(
        body,
        grid=(N // _R,),
        in_specs=[
            pl.BlockSpec((_R, D), lambda i: (i, 0)),
            pl.BlockSpec((_R, 1), lambda i: (i, 0)),
            pl.BlockSpec((D, D), lambda i: (0, 0)),
        ],
        out_specs=pl.BlockSpec((NC, _R, DH), lambda i: (0, i, 0)),
        out_shape=jax.ShapeDtypeStruct((NC, N, DH), jnp.float32),
    )(x, isd, W)


def _tc_mid(v, isd, b, W):
    def body(v_ref, i_ref, b_ref, w_ref, o_ref):
        vb = v_ref[...]
        hcat = jnp.concatenate([vb[0], vb[1]], axis=1)
        h = jnp.maximum(hcat * i_ref[...] + b_ref[...], 0.0)
        p = h * i_ref[...]
        ub = jnp.dot(p, w_ref[...], preferred_element_type=jnp.float32)
        o_ref[...] = jnp.stack([ub[:, :DH], ub[:, DH:]])

    return pl.pallas_call(
        body,
        grid=(N // _R,),
        in_specs=[
            pl.BlockSpec((NC, _R, DH), lambda i: (0, i, 0)),
            pl.BlockSpec((_R, 1), lambda i: (i, 0)),
            pl.BlockSpec((1, D), lambda i: (0, 0)),
            pl.BlockSpec((D, D), lambda i: (0, 0)),
        ],
        out_specs=pl.BlockSpec((NC, _R, DH), lambda i: (0, i, 0)),
        out_shape=jax.ShapeDtypeStruct((NC, N, DH), jnp.float32),
    )(v, isd, b, W)


def _tc_out(v, isd, b):
    def body(v_ref, i_ref, b_ref, o_ref):
        vb = v_ref[...]
        hcat = jnp.concatenate([vb[0], vb[1]], axis=1)
        o_ref[...] = jnp.maximum(hcat * i_ref[...] + b_ref[...], 0.0)

    return pl.pallas_call(
        body,
        grid=(N // _R,),
        in_specs=[
            pl.BlockSpec((NC, _R, DH), lambda i: (0, i, 0)),
            pl.BlockSpec((_R, 1), lambda i: (i, 0)),
            pl.BlockSpec((1, D), lambda i: (0, 0)),
        ],
        out_specs=pl.BlockSpec((_R, D), lambda i: (i, 0)),
        out_shape=jax.ShapeDtypeStruct((N, D), jnp.float32),
    )(v, isd, b)


@jax.jit
def kernel(x, edge_index, W1, b1, W2, b2, W3, b3):
    src = edge_index[0]
    dst = edge_index[1]
    srcp = jnp.concatenate([src, jnp.zeros((EPAD - E,), jnp.int32)])
    dstp = jnp.concatenate([dst, jnp.full((EPAD - E,), PAD_DST, jnp.int32)])
    src2d = srcp.reshape(EPAD // 128, 128)
    dst2d = dstp.reshape(EPAD // 128, 128)
    z1 = jnp.zeros((NPAD,), jnp.float32)
    z2 = jnp.zeros((NPAD, DH), jnp.float32)
    ones2d = jnp.ones((128,), jnp.float32)

    degp = _sc_degree(dst2d, z1, ones2d)
    isd = _tc_isd(degp).reshape(N, 1)

    u = _tc_in(x, isd, W1)
    v = _sc_spmm(u, srcp, dst2d, z2)
    u = _tc_mid(v, isd, b1.reshape(1, D), W2)
    v = _sc_spmm(u, srcp, dst2d, z2)
    u = _tc_mid(v, isd, b2.reshape(1, D), W3)
    v = _sc_spmm(u, srcp, dst2d, z2)
    return _tc_out(v, isd, b3.reshape(1, D))


# speedup vs baseline: 1.3344x; 1.1883x over previous
import functools

import jax
import jax.numpy as jnp
from jax import lax
from jax.experimental import pallas as pl
from jax.experimental.pallas import tpu as pltpu
from jax.experimental.pallas import tpu_sc as plsc

N = 10000
E = 160000
D = 256
DH = 128
NC, NS = 2, 16
NPAD = 10240
PAD_DST = 10048
EPAD = 163840
EPT = EPAD // NS
NCH = EPT // 128
CH = 5
DEG_NCH = EPAD // 32 // 128
STRIPE = NPAD // NS

_mesh = plsc.VectorSubcoreMesh(core_axis_name="c", subcore_axis_name="s")


@functools.partial(
    pl.kernel,
    out_type=jax.ShapeDtypeStruct((NC, NPAD), jnp.float32),
    mesh=_mesh,
    scratch_types=[
        pltpu.VMEM((DEG_NCH, 128), jnp.int32),
        pltpu.VMEM((128,), jnp.float32),
        pltpu.VMEM_SHARED((NPAD,), jnp.float32),
    ],
)
def _sc_degree(dst_hbm, z_hbm, ones_hbm, out_hbm, didx, ones_v, acc):
    c = lax.axis_index("c")
    s = lax.axis_index("s")
    tid = c * NS + s
    pltpu.sync_copy(z_hbm.at[pl.ds(s * STRIPE, STRIPE)],
                    acc.at[pl.ds(s * STRIPE, STRIPE)])
    pltpu.sync_copy(dst_hbm.at[pl.ds(tid * DEG_NCH, DEG_NCH)], didx)
    pltpu.sync_copy(ones_hbm, ones_v)
    plsc.subcore_barrier()

    @pl.loop(0, DEG_NCH)
    def _(j):
        pltpu.sync_copy(ones_v, acc.at[didx.at[j]], add=True)

    plsc.subcore_barrier()
    pltpu.sync_copy(acc.at[pl.ds(s * STRIPE, STRIPE)],
                    out_hbm.at[c].at[pl.ds(s * STRIPE, STRIPE)])


@functools.partial(
    pl.kernel,
    out_type=jax.ShapeDtypeStruct((NC, NPAD, DH), jnp.float32),
    mesh=_mesh,
    scratch_types=[
        pltpu.VMEM((EPT,), jnp.int32),
        pltpu.VMEM((NCH * 2, 64), jnp.int32),
        pltpu.VMEM((64, DH), jnp.float32),
        pltpu.VMEM((64, DH), jnp.float32),
        pltpu.VMEM_SHARED((NPAD, DH), jnp.float32),
        pltpu.SemaphoreType.DMA,
        pltpu.SemaphoreType.DMA,
    ],
)
def _sc_spmm(u_hbm, src_hbm, dst_hbm, z_hbm, dummy_hbm, out_hbm, sidx, didx,
             gbufa, gbufb, acc, gsema, gsemb):
    c = lax.axis_index("c")
    s = lax.axis_index("s")
    @pl.loop(0, STRIPE // 128)
    def _(r):
        pltpu.sync_copy(z_hbm.at[pl.ds(s * STRIPE + r * 128, 128)],
                        acc.at[pl.ds(s * STRIPE + r * 128, 128)])

    pltpu.sync_copy(src_hbm.at[pl.ds(s * EPT, EPT)], sidx)
    pltpu.sync_copy(dst_hbm.at[pl.ds(s * NCH * 2, NCH * 2)], didx)
    plsc.subcore_barrier()

    def gath(j, buf, sem):
        pltpu.async_copy(u_hbm.at[c].at[sidx.at[pl.ds(j * 64, 64)]],
                         buf, sem)

    def wait_g(buf, sem):
        pltpu.make_async_copy(dummy_hbm, buf, sem).wait()

    gath(0, gbufa, gsema)

    @pl.loop(0, NCH)
    def _(p):
        j = p * 2
        gath(j + 1, gbufb, gsemb)
        wait_g(gbufa, gsema)
        pltpu.sync_copy(gbufa, acc.at[didx.at[j]], add=True)

        @pl.when(p < NCH - 1)
        def _():
            gath(j + 2, gbufa, gsema)

        wait_g(gbufb, gsemb)
        pltpu.sync_copy(gbufb, acc.at[didx.at[j + 1]], add=True)

    plsc.subcore_barrier()

    @pl.loop(0, STRIPE // 128)
    def _(r):
        pltpu.sync_copy(acc.at[pl.ds(s * STRIPE + r * 128, 128)],
                        out_hbm.at[c].at[pl.ds(s * STRIPE + r * 128, 128)])


_R = 1000


def _tc_isd(degp):
    def body(d_ref, o_ref):
        d = d_ref[0:1, :N] + d_ref[1:2, :N]
        o_ref[...] = lax.rsqrt(jnp.maximum(d, 1.0))

    return pl.pallas_call(
        body,
        out_shape=jax.ShapeDtypeStruct((1, N), jnp.float32),
    )(degp)


def _tc_in(x, isd, W):
    def body(x_ref, i_ref, w_ref, o_ref):
        p = x_ref[...] * i_ref[...]
        ub = jnp.dot(p, w_ref[...], preferred_element_type=jnp.float32)
        o_ref[...] = jnp.stack([ub[:, :DH], ub[:, DH:]])

    return pl.pallas_call(
        body,
        grid=(N // _R,),
        in_specs=[
            pl.BlockSpec((_R, D), lambda i: (i, 0)),
            pl.BlockSpec((_R, 1), lambda i: (i, 0)),
            pl.BlockSpec((D, D), lambda i: (0, 0)),
        ],
        out_specs=pl.BlockSpec((NC, _R, DH), lambda i: (0, i, 0)),
        out_shape=jax.ShapeDtypeStruct((NC, N, DH), jnp.float32),
    )(x, isd, W)


def _tc_mid(v, isd, b, W):
    def body(v_ref, i_ref, b_ref, w_ref, o_ref):
        vb = v_ref[...]
        hcat = jnp.concatenate([vb[0], vb[1]], axis=1)
        h = jnp.maximum(hcat * i_ref[...] + b_ref[...], 0.0)
        p = h * i_ref[...]
        ub = jnp.dot(p, w_ref[...], preferred_element_type=jnp.float32)
        o_ref[...] = jnp.stack([ub[:, :DH], ub[:, DH:]])

    return pl.pallas_call(
        body,
        grid=(N // _R,),
        in_specs=[
            pl.BlockSpec((NC, _R, DH), lambda i: (0, i, 0)),
            pl.BlockSpec((_R, 1), lambda i: (i, 0)),
            pl.BlockSpec((1, D), lambda i: (0, 0)),
            pl.BlockSpec((D, D), lambda i: (0, 0)),
        ],
        out_specs=pl.BlockSpec((NC, _R, DH), lambda i: (0, i, 0)),
        out_shape=jax.ShapeDtypeStruct((NC, N, DH), jnp.float32),
    )(v, isd, b, W)


def _tc_out(v, isd, b):
    def body(v_ref, i_ref, b_ref, o_ref):
        vb = v_ref[...]
        hcat = jnp.concatenate([vb[0], vb[1]], axis=1)
        o_ref[...] = jnp.maximum(hcat * i_ref[...] + b_ref[...], 0.0)

    return pl.pallas_call(
        body,
        grid=(N // _R,),
        in_specs=[
            pl.BlockSpec((NC, _R, DH), lambda i: (0, i, 0)),
            pl.BlockSpec((_R, 1), lambda i: (i, 0)),
            pl.BlockSpec((1, D), lambda i: (0, 0)),
        ],
        out_specs=pl.BlockSpec((_R, D), lambda i: (i, 0)),
        out_shape=jax.ShapeDtypeStruct((N, D), jnp.float32),
    )(v, isd, b)


@jax.jit
def kernel(x, edge_index, W1, b1, W2, b2, W3, b3):
    src = edge_index[0]
    dst = edge_index[1]
    srcp = jnp.concatenate([src, jnp.zeros((EPAD - E,), jnp.int32)])
    dstp = jnp.concatenate([dst, jnp.full((EPAD - E,), PAD_DST, jnp.int32)])
    src2d = srcp.reshape(EPAD // 128, 128)
    dst2d = dstp.reshape(EPAD // 128, 128)
    dst2d64 = dstp.reshape(EPAD // 64, 64)
    z1 = jnp.zeros((NPAD,), jnp.float32)
    z2 = jnp.zeros((NPAD, DH), jnp.float32)
    ones2d = jnp.ones((128,), jnp.float32)
    dummy64 = jnp.zeros((64, DH), jnp.float32)

    degp = _sc_degree(dst2d, z1, ones2d)
    isd = _tc_isd(degp).reshape(N, 1)

    u = _tc_in(x, isd, W1)
    v = _sc_spmm(u, srcp, dst2d64, z2, dummy64)
    u = _tc_mid(v, isd, b1.reshape(1, D), W2)
    v = _sc_spmm(u, srcp, dst2d64, z2, dummy64)
    u = _tc_mid(v, isd, b2.reshape(1, D), W3)
    v = _sc_spmm(u, srcp, dst2d64, z2, dummy64)
    return _tc_out(v, isd, b3.reshape(1, D))
